# baseline (speedup 1.0000x reference)
import functools

import jax
import jax.numpy as jnp
from jax import lax
from jax.experimental import pallas as pl
from jax.experimental.pallas import tpu as pltpu
from jax.experimental.pallas import tpu_sc as plsc

BATCH = 16384
IN_DIM = 128
EMBED = 64
VOCAB = 1000000

CW = 512
NFULL = VOCAB // CW
TAILW = VOCAB - NFULL * CW
RING = 8
SROWS = BATCH


def _make_sc_sweep():
    info = plsc.get_sparse_core_info()
    NC, NS = info.num_cores, info.num_subcores
    NW = NC * NS
    mesh = plsc.VectorSubcoreMesh(core_axis_name="c", subcore_axis_name="s")
    iota16 = lambda: lax.iota(jnp.int32, 16)

    @functools.partial(
        pl.kernel,
        mesh=mesh,
        out_type=jax.ShapeDtypeStruct((SROWS, 128), jnp.float32),
        compiler_params=pltpu.CompilerParams(needs_layout_passes=False),
        scratch_types=[
            pltpu.VMEM((2048,), jnp.int32),
            pltpu.VMEM((BATCH + 16,), jnp.int32),
            pltpu.VMEM((BATCH + 16,), jnp.int32),
            pltpu.VMEM((EMBED, CW), jnp.float32),
            pltpu.VMEM((EMBED, CW), jnp.float32),
            pltpu.VMEM((48,), jnp.int32),
            pltpu.VMEM((48,), jnp.int32),
            pltpu.VMEM((RING, 16, 128), jnp.float32),
            pltpu.VMEM((RING, 16), jnp.int32),
            pltpu.VMEM((EMBED, TAILW), jnp.float32),
            pltpu.SemaphoreType.DMA,
            pltpu.SemaphoreType.DMA,
        ],
    )
    def sweep_kernel(tabT_hbm, tailT_hbm, idx_hbm, out_hbm, idx_v, ci_v,
                     cb_v, ch0_v, ch1_v, hr_v, hb_v, rg_v, bt_v, tail_v,
                     csem, ssem):
        wid = lax.axis_index("s") * NC + lax.axis_index("c")
        nt = (NFULL - wid + NW - 1) // NW

        def issue(t, buf_ref):
            c = wid + t * NW
            pltpu.async_copy(
                tabT_hbm.at[:, pl.ds(c * CW, CW)], buf_ref, csem)

        @pl.when(nt > 0)
        def _():
            issue(0, ch0_v)

        @pl.when(nt > 1)
        def _():
            issue(1, ch1_v)

        def scan_outer(ob, off):
            pltpu.sync_copy(idx_hbm.at[pl.ds(ob * 2048, 2048)], idx_v)

            def scan_body(i, off):
                iv = idx_v[pl.ds(i * 16, 16)]
                m = ((iv >> 9) & (NW - 1)) == wid
                cnt = plsc.all_reduce_population_count(m)[0]

                @pl.when(cnt > 0)
                def _():
                    key = jnp.where(m, iota16(), iota16() + 16)
                    _, iv_c = plsc.sort_key_val(key, iv)
                    ci_v[pl.ds(off, 16)] = iv_c
                    bv = ob * 2048 + i * 16 + iota16()
                    _, bv_c = plsc.sort_key_val(key, bv)
                    cb_v[pl.ds(off, 16)] = bv_c

                return off + cnt

            return lax.fori_loop(0, 128, scan_body, off)

        ncand = lax.fori_loop(0, BATCH // 2048, scan_outer, 0)

        def emit(chunk_ref, hoff, ring, nfl):
            @pl.when(nfl >= RING)
            def _():
                pltpu.make_async_copy(
                    rg_v.at[0], out_hbm.at[bt_v.at[0]], ssem).wait()

            hv = hr_v[pl.ds(0, 16)]
            hbv = hb_v[pl.ds(0, 16)]
            valid = iota16() < hoff
            hv = jnp.where(valid, hv, jnp.broadcast_to(hv[0], (16,)))
            bvec = jnp.where(valid, hbv, jnp.broadcast_to(hbv[0], (16,)))
            bt_v[ring, :] = bvec

            def dgather(d, carry):
                vals = plsc.load_gather(
                    chunk_ref, [jnp.broadcast_to(d, (16,)), hv])
                plsc.store_scatter(
                    rg_v, [jnp.broadcast_to(ring, (16,)), iota16(),
                           jnp.broadcast_to(d, (16,))], vals)
                return carry

            lax.fori_loop(0, EMBED, dgather, 0)

            pltpu.async_copy(rg_v.at[ring], out_hbm.at[bt_v.at[ring]], ssem)

            hr_v[pl.ds(0, 16)] = hr_v[pl.ds(16, 16)]
            hb_v[pl.ds(0, 16)] = hb_v[pl.ds(16, 16)]
            return (jnp.maximum(hoff - 16, 0), (ring + 1) & (RING - 1),
                    jnp.minimum(nfl + 1, RING))

        def process(chunk_ref, c, ring, nfl):
            def cscan(v, carry):
                hoff, ring, nfl = carry
                cv = ci_v[pl.ds(v * 16, 16)]
                bv = cb_v[pl.ds(v * 16, 16)]
                inb = (v * 16 + iota16()) < ncand
                m = ((cv >> 9) == c) & inb
                key = jnp.where(m, iota16(), iota16() + 16)
                _, rv_c = plsc.sort_key_val(key, cv & (CW - 1))
                hr_v[pl.ds(hoff, 16)] = rv_c
                _, bv_c = plsc.sort_key_val(key, bv)
                hb_v[pl.ds(hoff, 16)] = bv_c
                hoff = hoff + plsc.all_reduce_population_count(m)[0]

                def flush(carry):
                    return emit(chunk_ref, *carry)

                hoff, ring, nfl = lax.cond(
                    hoff >= 16, flush, lambda carry: carry,
                    (hoff, ring, nfl))
                return (hoff, ring, nfl)

            nv = (ncand + 15) >> 4
            hoff, ring, nfl = lax.fori_loop(0, nv, cscan, (0, ring, nfl))

            def flush2(carry):
                return emit(chunk_ref, *carry)

            hoff, ring, nfl = lax.cond(
                hoff > 0, flush2, lambda carry: carry, (hoff, ring, nfl))
            return ring, nfl

        def wait_chunk():
            pltpu.make_async_copy(
                tabT_hbm.at[:, pl.ds(0, CW)], ch0_v, csem).wait()

        def pair_body(t2, carry):
            ring, nfl = carry
            for k, cur in enumerate((ch0_v, ch1_v)):
                t = t2 * 2 + k

                @pl.when(t < nt)
                def _():
                    wait_chunk()

                def do_proc(carry):
                    ring, nfl = carry
                    return process(cur, wid + t * NW, ring, nfl)

                ring, nfl = lax.cond(
                    t < nt, do_proc, lambda carry: carry, (ring, nfl))

                @pl.when(t + 2 < nt)
                def _():
                    issue(t + 2, cur)
            return (ring, nfl)

        ring, nfl = lax.fori_loop(0, (nt + 1) // 2, pair_body, (0, 0))

        @pl.when(wid == (NFULL % NW))
        def _():
            pltpu.sync_copy(tailT_hbm, tail_v)

        tail_ring, tail_nfl = lax.cond(
            wid == (NFULL % NW),
            lambda carry: process(tail_v, NFULL, *carry),
            lambda carry: carry, (ring, nfl))

        def drain_body(i, carry):
            pltpu.make_async_copy(
                rg_v.at[0], out_hbm.at[bt_v.at[0]], ssem).wait()
            return carry

        lax.fori_loop(0, tail_nfl, drain_body, 0)

    return sweep_kernel


def _tc_paper_body(pf_ref, wp_ref, opT_ref):
    opT_ref[...] = lax.dot_general(
        wp_ref[...], pf_ref[...], (((0,), (1,)), ((), ())),
        preferred_element_type=jnp.float32)


def _tc_author_body(ga_ref, wa_ref, oaT_ref):
    ne = ga_ref[...][:, :EMBED]
    oaT_ref[...] = lax.dot_general(
        wa_ref[...], ne, (((0,), (1,)), ((), ())),
        preferred_element_type=jnp.float32)


@jax.jit
def kernel(paper_feats, paper_nodes, author_nodes, W_paper_proj,
           author_embed_table, W_author_proj):
    tabT = jnp.transpose(author_embed_table)
    tailT = jnp.transpose(author_embed_table[NFULL * CW:])
    idx32 = author_nodes.astype(jnp.int32)
    gathered = _make_sc_sweep()(tabT, tailT, idx32)

    BLK = 2048
    nb = BATCH // BLK
    opT = pl.pallas_call(
        _tc_paper_body,
        grid=(nb,),
        in_specs=[
            pl.BlockSpec((BLK, IN_DIM), lambda i: (i, 0)),
            pl.BlockSpec((IN_DIM, EMBED), lambda i: (0, 0)),
        ],
        out_specs=pl.BlockSpec((EMBED, BLK), lambda i: (0, i)),
        out_shape=jax.ShapeDtypeStruct((EMBED, BATCH), jnp.float32),
    )(paper_feats, W_paper_proj)
    oaT = pl.pallas_call(
        _tc_author_body,
        grid=(nb,),
        in_specs=[
            pl.BlockSpec((BLK, 128), lambda i: (i, 0)),
            pl.BlockSpec((EMBED, EMBED), lambda i: (0, 0)),
        ],
        out_specs=pl.BlockSpec((EMBED, BLK), lambda i: (0, i)),
        out_shape=jax.ShapeDtypeStruct((EMBED, BATCH), jnp.float32),
    )(gathered, W_author_proj)
    return (jnp.transpose(opT), jnp.transpose(oaT))


# speedup vs baseline: 1.0569x; 1.0569x over previous
import functools

import jax
import jax.numpy as jnp
from jax import lax
from jax.experimental import pallas as pl
from jax.experimental.pallas import tpu as pltpu
from jax.experimental.pallas import tpu_sc as plsc

BATCH = 16384
IN_DIM = 128
EMBED = 64
VOCAB = 1000000

CW = 512
NFULL = VOCAB // CW
TAILW = VOCAB - NFULL * CW
RING = 8
SROWS = BATCH


def _make_sc_sweep():
    info = plsc.get_sparse_core_info()
    NC, NS = info.num_cores, info.num_subcores
    NW = NC * NS
    mesh = plsc.VectorSubcoreMesh(core_axis_name="c", subcore_axis_name="s")
    iota16 = lambda: lax.iota(jnp.int32, 16)

    @functools.partial(
        pl.kernel,
        mesh=mesh,
        out_type=jax.ShapeDtypeStruct((SROWS, 128), jnp.float32),
        compiler_params=pltpu.CompilerParams(needs_layout_passes=False),
        scratch_types=[
            pltpu.VMEM((2048,), jnp.int32),
            pltpu.VMEM((BATCH + 16,), jnp.int32),
            pltpu.VMEM((BATCH + 16,), jnp.int32),
            pltpu.VMEM((EMBED, CW), jnp.float32),
            pltpu.VMEM((EMBED, CW), jnp.float32),
            pltpu.VMEM((48,), jnp.int32),
            pltpu.VMEM((48,), jnp.int32),
            pltpu.VMEM((RING, 16, 128), jnp.float32),
            pltpu.VMEM((RING, 16), jnp.int32),
            pltpu.VMEM((EMBED, TAILW), jnp.float32),
            pltpu.SemaphoreType.DMA,
            pltpu.SemaphoreType.DMA,
        ],
    )
    def sweep_kernel(tabT_hbm, tailT_hbm, idx_hbm, out_hbm, idx_v, ci_v,
                     cb_v, ch0_v, ch1_v, hr_v, hb_v, rg_v, bt_v, tail_v,
                     csem, ssem):
        wid = lax.axis_index("s") * NC + lax.axis_index("c")

        def scan_outer(ob, off):
            pltpu.sync_copy(idx_hbm.at[pl.ds(ob * 2048, 2048)], idx_v)

            def scan_body(i, off):
                iv = idx_v[pl.ds(i * 16, 16)]
                m = ((iv >> 9) & (NW - 1)) == wid
                key = jnp.where(m, iota16(), iota16() + 16)
                _, iv_c = plsc.sort_key_val(key, iv)
                ci_v[pl.ds(off, 16)] = iv_c
                bv = ob * 2048 + i * 16 + iota16()
                _, bv_c = plsc.sort_key_val(key, bv)
                cb_v[pl.ds(off, 16)] = bv_c
                return off + plsc.all_reduce_population_count(m)[0]

            return lax.fori_loop(0, 128, scan_body, off)

        ncand = lax.fori_loop(0, BATCH // 2048, scan_outer, 0)

        def emit(chunk_ref, hoff, ring, nfl):
            @pl.when(nfl >= RING)
            def _():
                pltpu.make_async_copy(
                    rg_v.at[0], out_hbm.at[bt_v.at[0]], ssem).wait()

            hv = hr_v[pl.ds(0, 16)]
            hbv = hb_v[pl.ds(0, 16)]
            valid = iota16() < hoff
            hv = jnp.where(valid, hv, jnp.broadcast_to(hv[0], (16,)))
            bvec = jnp.where(valid, hbv, jnp.broadcast_to(hbv[0], (16,)))
            bt_v[ring, :] = bvec

            def dgather(d, carry):
                vals = plsc.load_gather(
                    chunk_ref, [jnp.broadcast_to(d, (16,)), hv])
                plsc.store_scatter(
                    rg_v, [jnp.broadcast_to(ring, (16,)), iota16(),
                           jnp.broadcast_to(d, (16,))], vals)
                return carry

            lax.fori_loop(0, EMBED, dgather, 0)

            pltpu.async_copy(rg_v.at[ring], out_hbm.at[bt_v.at[ring]], ssem)

            hr_v[pl.ds(0, 16)] = hr_v[pl.ds(16, 16)]
            hb_v[pl.ds(0, 16)] = hb_v[pl.ds(16, 16)]
            return (jnp.maximum(hoff - 16, 0), (ring + 1) & (RING - 1),
                    jnp.minimum(nfl + 1, RING))

        def process(chunk_ref, c, ring, nfl):
            def cscan(v, carry):
                hoff, ring, nfl = carry
                cv = ci_v[pl.ds(v * 16, 16)]
                bv = cb_v[pl.ds(v * 16, 16)]
                inb = (v * 16 + iota16()) < ncand
                m = ((cv >> 9) == c) & inb
                key = jnp.where(m, iota16(), iota16() + 16)
                _, rv_c = plsc.sort_key_val(key, cv & (CW - 1))
                hr_v[pl.ds(hoff, 16)] = rv_c
                _, bv_c = plsc.sort_key_val(key, bv)
                hb_v[pl.ds(hoff, 16)] = bv_c
                hoff = hoff + plsc.all_reduce_population_count(m)[0]

                def flush(carry):
                    return emit(chunk_ref, *carry)

                hoff, ring, nfl = lax.cond(
                    hoff >= 16, flush, lambda carry: carry,
                    (hoff, ring, nfl))
                return (hoff, ring, nfl)

            nv = (ncand + 15) >> 4
            hoff, ring, nfl = lax.fori_loop(0, nv, cscan, (0, ring, nfl))

            def flush2(carry):
                return emit(chunk_ref, *carry)

            hoff, ring, nfl = lax.cond(
                hoff > 0, flush2, lambda carry: carry, (hoff, ring, nfl))
            return ring, nfl

        nt = (NFULL - wid + NW - 1) // NW

        def issue(t, buf_ref):
            c = wid + t * NW
            pltpu.async_copy(
                tabT_hbm.at[:, pl.ds(c * CW, CW)], buf_ref, csem)

        def wait_chunk():
            pltpu.make_async_copy(
                tabT_hbm.at[:, pl.ds(0, CW)], ch0_v, csem).wait()

        @pl.when(nt > 0)
        def _():
            issue(0, ch0_v)

        @pl.when(nt > 1)
        def _():
            issue(1, ch1_v)

        def pair_body(t2, carry):
            ring, nfl = carry
            for k, cur in enumerate((ch0_v, ch1_v)):
                t = t2 * 2 + k

                @pl.when(t < nt)
                def _():
                    wait_chunk()

                def do_proc(carry):
                    ring, nfl = carry
                    return process(cur, wid + t * NW, ring, nfl)

                ring, nfl = lax.cond(
                    t < nt, do_proc, lambda carry: carry, (ring, nfl))

                @pl.when(t + 2 < nt)
                def _():
                    issue(t + 2, cur)
            return (ring, nfl)

        ring, nfl = lax.fori_loop(0, (nt + 1) // 2, pair_body, (0, 0))

        @pl.when(wid == (NFULL % NW))
        def _():
            pltpu.sync_copy(tailT_hbm, tail_v)

        tail_ring, tail_nfl = lax.cond(
            wid == (NFULL % NW),
            lambda carry: process(tail_v, NFULL, *carry),
            lambda carry: carry, (ring, nfl))

        def drain_body(i, carry):
            pltpu.make_async_copy(
                rg_v.at[0], out_hbm.at[bt_v.at[0]], ssem).wait()
            return carry

        lax.fori_loop(0, tail_nfl, drain_body, 0)

    return sweep_kernel


def _tc_paper_body(pf_ref, wp_ref, opT_ref):
    opT_ref[...] = lax.dot_general(
        wp_ref[...], pf_ref[...], (((0,), (1,)), ((), ())),
        preferred_element_type=jnp.float32)


def _tc_author_body(ga_ref, wa_ref, oaT_ref):
    ne = ga_ref[...][:, :EMBED]
    oaT_ref[...] = lax.dot_general(
        wa_ref[...], ne, (((0,), (1,)), ((), ())),
        preferred_element_type=jnp.float32)


@jax.jit
def kernel(paper_feats, paper_nodes, author_nodes, W_paper_proj,
           author_embed_table, W_author_proj):
    tabT = jnp.transpose(author_embed_table)
    tailT = jnp.transpose(author_embed_table[NFULL * CW:])
    idx32 = author_nodes.astype(jnp.int32)
    gathered = _make_sc_sweep()(tabT, tailT, idx32)

    BLK = 2048
    nb = BATCH // BLK
    opT = pl.pallas_call(
        _tc_paper_body,
        grid=(nb,),
        in_specs=[
            pl.BlockSpec((BLK, IN_DIM), lambda i: (i, 0)),
            pl.BlockSpec((IN_DIM, EMBED), lambda i: (0, 0)),
        ],
        out_specs=pl.BlockSpec((EMBED, BLK), lambda i: (0, i)),
        out_shape=jax.ShapeDtypeStruct((EMBED, BATCH), jnp.float32),
    )(paper_feats, W_paper_proj)
    oaT = pl.pallas_call(
        _tc_author_body,
        grid=(nb,),
        in_specs=[
            pl.BlockSpec((BLK, 128), lambda i: (i, 0)),
            pl.BlockSpec((EMBED, EMBED), lambda i: (0, 0)),
        ],
        out_specs=pl.BlockSpec((EMBED, BLK), lambda i: (0, i)),
        out_shape=jax.ShapeDtypeStruct((EMBED, BATCH), jnp.float32),
    )(gathered, W_author_proj)
    return (jnp.transpose(opT), jnp.transpose(oaT))
